# baseline (speedup 1.0000x reference)
import jax
import jax.numpy as jnp
from jax.experimental import pallas as pl

N_ATTRIBUTES = 26
N_VALUES = 128
LEVEL = 13
D = N_ATTRIBUTES * N_VALUES
KEEP = LEVEL * N_VALUES
ZERO = D - KEEP
BM = 1024


def _mask_kernel(x_ref, out_ref, idx_ref, nrev_ref):
    out_ref[:, :KEEP] = x_ref[...]
    out_ref[:, KEEP:] = jnp.zeros((x_ref.shape[0], ZERO), x_ref.dtype)
    idx_ref[...] = jax.lax.broadcasted_iota(jnp.int32, idx_ref.shape, 1)
    nrev_ref[...] = jnp.full(nrev_ref.shape, LEVEL, jnp.int32)


def kernel(sender_input, labels):
    B = sender_input.shape[0]
    grid = (B // BM,)
    masked, idxs_to_reveal, n_revealed = pl.pallas_call(
        _mask_kernel,
        grid=grid,
        in_specs=[pl.BlockSpec((BM, KEEP), lambda i: (i, 0))],
        out_specs=[
            pl.BlockSpec((BM, D), lambda i: (i, 0)),
            pl.BlockSpec((BM, LEVEL), lambda i: (i, 0)),
            pl.BlockSpec((BM, 1), lambda i: (i, 0)),
        ],
        out_shape=[
            jax.ShapeDtypeStruct((B, D), sender_input.dtype),
            jax.ShapeDtypeStruct((B, LEVEL), jnp.int32),
            jax.ShapeDtypeStruct((B, 1), jnp.int32),
        ],
    )(sender_input)
    return masked, idxs_to_reveal, n_revealed.reshape(B)


# speedup vs baseline: 1.0468x; 1.0468x over previous
import jax
import jax.numpy as jnp
from jax.experimental import pallas as pl

N_ATTRIBUTES = 26
N_VALUES = 128
LEVEL = 13
D = N_ATTRIBUTES * N_VALUES
KEEP = LEVEL * N_VALUES
ZERO = D - KEEP
BM = 1024


def _mask_kernel(x_ref, out_ref):
    j = pl.program_id(1)

    @pl.when(j == 0)
    def _copy():
        out_ref[...] = x_ref[...]

    @pl.when(j == 1)
    def _zero():
        out_ref[...] = jnp.zeros_like(out_ref)


def kernel(sender_input, labels):
    B = sender_input.shape[0]
    grid = (B // BM, 2)
    masked = pl.pallas_call(
        _mask_kernel,
        grid=grid,
        in_specs=[pl.BlockSpec((BM, KEEP), lambda i, j: (i, 0))],
        out_specs=pl.BlockSpec((BM, KEEP), lambda i, j: (i, j)),
        out_shape=jax.ShapeDtypeStruct((B, D), sender_input.dtype),
    )(sender_input)
    idxs_to_reveal = jnp.broadcast_to(
        jnp.arange(LEVEL, dtype=jnp.int32), (B, LEVEL)
    )
    n_revealed = jnp.full((B,), LEVEL, dtype=jnp.int32)
    return masked, idxs_to_reveal, n_revealed


# speedup vs baseline: 1.1417x; 1.0907x over previous
import jax
import jax.numpy as jnp
from jax.experimental import pallas as pl

N_ATTRIBUTES = 26
N_VALUES = 128
LEVEL = 13
D = N_ATTRIBUTES * N_VALUES
KEEP = LEVEL * N_VALUES
ZERO = D - KEEP
BM = 1024


def _mask_kernel(x_ref, out_ref):
    out_ref[:, :KEEP] = x_ref[...]
    out_ref[:, KEEP:] = jnp.zeros((x_ref.shape[0], ZERO), x_ref.dtype)


def kernel(sender_input, labels):
    B = sender_input.shape[0]
    grid = (B // BM,)
    masked = pl.pallas_call(
        _mask_kernel,
        grid=grid,
        in_specs=[pl.BlockSpec((BM, KEEP), lambda i: (i, 0))],
        out_specs=pl.BlockSpec((BM, D), lambda i: (i, 0)),
        out_shape=jax.ShapeDtypeStruct((B, D), sender_input.dtype),
    )(sender_input)
    idxs_to_reveal = jnp.broadcast_to(
        jnp.arange(LEVEL, dtype=jnp.int32), (B, LEVEL)
    )
    n_revealed = jnp.full((B,), LEVEL, dtype=jnp.int32)
    return masked, idxs_to_reveal, n_revealed
